# baseline (device time: 65730 ns/iter reference)
import jax
import jax.numpy as jnp
from jax import lax
from jax.experimental import pallas as pl
from jax.experimental.pallas import tpu as pltpu

N_DEV = 4
ORDER = (2, 1, 3)
SUB = 2
I16MAX = 32767.0


def kernel(x, w_mat):
    m_per, k = x.shape
    _, n = w_mat.shape
    n_per = n // N_DEV
    n_sub = n_per // SUB
    NPEER = (N_DEV - 1) * SUB

    def body(x_ref, w_ref, out_ref, w_buf, iout_ref, iin_ref,
             amax_send, amax_recv, w_sems, send_sems, recv_sems,
             am_send_sems, am_recv_sems):
        my = lax.axis_index("i")

        barrier_sem = pltpu.get_barrier_semaphore()
        for d in range(1, N_DEV):
            pl.semaphore_signal(
                barrier_sem, inc=1,
                device_id=((my + d) % N_DEV,),
                device_id_type=pl.DeviceIdType.MESH,
            )
        pl.semaphore_wait(barrier_sem, N_DEV - 1)

        sched = [(d, c) for d in ORDER for c in range(SUB)]
        sched += [(0, c) for c in range(SUB)]
        n_steps = len(sched)

        def w_fetch(t):
            d, c = sched[t]
            col = ((my + d) % N_DEV) * n_per + c * n_sub
            cp = pltpu.make_async_copy(
                w_ref.at[:, pl.ds(col, n_sub)],
                w_buf.at[t % 2],
                w_sems.at[t % 2],
            )
            cp.start()
            return cp

        fetches = [w_fetch(0)]
        descs = []
        amax = jnp.float32(0.0)
        submax = {}
        for t in range(n_steps):
            if t + 1 < n_steps:
                fetches.append(w_fetch(t + 1))
            fetches[t].wait()
            blk = jnp.dot(
                x_ref[...], w_buf[t % 2],
                preferred_element_type=jnp.float32,
            )
            d, c = sched[t]
            if d:
                s = (d - 1) * SUB + c
                bm = jnp.maximum(jnp.max(jnp.abs(blk)), jnp.float32(1e-30))
                submax[s] = bm
                amax = jnp.maximum(amax, bm)
                iout_ref[d - 1, :, pl.ds(c * n_sub, n_sub)] = jnp.round(
                    blk * (I16MAX / bm)
                ).astype(jnp.int16)
                rdma = pltpu.make_async_remote_copy(
                    src_ref=iout_ref.at[d - 1, :, pl.ds(c * n_sub, n_sub)],
                    dst_ref=iin_ref.at[d - 1, :, pl.ds(c * n_sub, n_sub)],
                    send_sem=send_sems.at[s],
                    recv_sem=recv_sems.at[s],
                    device_id=((my + d) % N_DEV,),
                    device_id_type=pl.DeviceIdType.MESH,
                )
                rdma.start()
                descs.append(rdma)
            else:
                amax = jnp.maximum(amax, jnp.max(jnp.abs(blk)))
                out_ref[pl.ds(my * m_per, m_per), pl.ds(c * n_sub, n_sub)] = blk

        col = lax.broadcasted_iota(jnp.int32, amax_send.shape, 1)
        msg = jnp.full(amax_send.shape, amax, jnp.float32)
        for s in range(NPEER):
            msg = jnp.where(col == s + 1, submax[s], msg)
        amax_send[...] = msg
        am_descs = []
        for d in range(1, N_DEV):
            rdma = pltpu.make_async_remote_copy(
                src_ref=amax_send,
                dst_ref=amax_recv.at[d - 1],
                send_sem=am_send_sems.at[d - 1],
                recv_sem=am_recv_sems.at[d - 1],
                device_id=((my + d) % N_DEV,),
                device_id_type=pl.DeviceIdType.MESH,
            )
            rdma.start()
            am_descs.append(rdma)
        for am in am_descs:
            am.wait_recv()

        gmax = jnp.maximum(amax, jnp.max(amax_recv[...]))
        scale = gmax / 448.0
        inv = 448.0 / gmax

        own = out_ref[pl.ds(my * m_per, m_per), :]
        out_ref[pl.ds(my * m_per, m_per), :] = (
            (own * inv).astype(jnp.float8_e4m3fn).astype(jnp.float32) * scale
        )

        for t in range(NPEER):
            d, c = sched[t]
            s = (d - 1) * SUB + c
            descs[t].wait_recv()
            src = (my - d) % N_DEV
            bm = jnp.max(amax_recv[d - 1, :, s + 1])
            val = iin_ref[d - 1, :, pl.ds(c * n_sub, n_sub)].astype(
                jnp.float32
            ) * (bm / I16MAX)
            out_ref[pl.ds(src * m_per, m_per), pl.ds(c * n_sub, n_sub)] = (
                (val * inv).astype(jnp.float8_e4m3fn).astype(jnp.float32)
                * scale
            )

        for rdma in descs + am_descs:
            rdma.wait_send()

    grid_spec = pltpu.PrefetchScalarGridSpec(
        num_scalar_prefetch=0,
        in_specs=[
            pl.BlockSpec(memory_space=pltpu.MemorySpace.VMEM),
            pl.BlockSpec(memory_space=pltpu.MemorySpace.HBM),
        ],
        out_specs=pl.BlockSpec(memory_space=pltpu.MemorySpace.VMEM),
        scratch_shapes=[
            pltpu.VMEM((2, k, n_sub), jnp.float32),
            pltpu.VMEM((N_DEV - 1, m_per, n_per), jnp.int16),
            pltpu.VMEM((N_DEV - 1, m_per, n_per), jnp.int16),
            pltpu.VMEM((8, 128), jnp.float32),
            pltpu.VMEM((N_DEV - 1, 8, 128), jnp.float32),
            pltpu.SemaphoreType.DMA((2,)),
            pltpu.SemaphoreType.DMA((NPEER,)),
            pltpu.SemaphoreType.DMA((NPEER,)),
            pltpu.SemaphoreType.DMA((N_DEV - 1,)),
            pltpu.SemaphoreType.DMA((N_DEV - 1,)),
        ],
    )
    return pl.pallas_call(
        body,
        out_shape=jax.ShapeDtypeStruct((N_DEV * m_per, n_per), jnp.float32),
        grid_spec=grid_spec,
        compiler_params=pltpu.CompilerParams(
            collective_id=0, vmem_limit_bytes=100 * 1024 * 1024
        ),
    )(x, w_mat)


# device time: 53390 ns/iter; 1.2311x vs baseline; 1.2311x over previous
import os

import jax
import jax.numpy as jnp
from jax import lax
from jax.experimental import pallas as pl
from jax.experimental.pallas import tpu as pltpu

N_DEV = 4
ORDER = (2, 1, 3)
I16MAX = 32767.0
OUT_HBM = os.environ.get("OUT_HBM", "1") == "1"


def kernel(x, w_mat):
    m_per, k = x.shape
    _, n = w_mat.shape
    n_per = n // N_DEV

    def body(x_ref, w_ref, out_ref, ov_ref, w_buf, iout_ref, iin_ref,
             amax_send, amax_recv, w_sems, ov_sems, send_sems, recv_sems,
             am_send_sems, am_recv_sems):
        my = lax.axis_index("i")
        ov = ov_ref if OUT_HBM else out_ref

        barrier_sem = pltpu.get_barrier_semaphore()
        for d in range(1, N_DEV):
            pl.semaphore_signal(
                barrier_sem, inc=1,
                device_id=((my + d) % N_DEV,),
                device_id_type=pl.DeviceIdType.MESH,
            )
        pl.semaphore_wait(barrier_sem, N_DEV - 1)

        offs = list(ORDER) + [0]

        def w_fetch(t):
            cp = pltpu.make_async_copy(
                w_ref.at[:, pl.ds(((my + offs[t]) % N_DEV) * n_per, n_per)],
                w_buf.at[t % 2],
                w_sems.at[t % 2],
            )
            cp.start()
            return cp

        out_dmas = []

        def flush(row):
            if OUT_HBM:
                cp = pltpu.make_async_copy(
                    ov_ref.at[pl.ds(row * m_per, m_per), :],
                    out_ref.at[pl.ds(row * m_per, m_per), :],
                    ov_sems.at[len(out_dmas)],
                )
                cp.start()
                out_dmas.append(cp)

        fetches = [w_fetch(0)]
        descs = []
        amax = jnp.float32(0.0)
        blkmax = {}
        for t in range(N_DEV):
            if t + 1 < N_DEV:
                fetches.append(w_fetch(t + 1))
            fetches[t].wait()
            blk = jnp.dot(
                x_ref[...], w_buf[t % 2],
                preferred_element_type=jnp.float32,
            )
            if t + 1 < N_DEV:
                d = offs[t]
                bm = jnp.maximum(jnp.max(jnp.abs(blk)), jnp.float32(1e-30))
                blkmax[d] = bm
                amax = jnp.maximum(amax, bm)
                iout_ref[d - 1] = jnp.round(blk * (I16MAX / bm)).astype(
                    jnp.int16
                )
                rdma = pltpu.make_async_remote_copy(
                    src_ref=iout_ref.at[d - 1],
                    dst_ref=iin_ref.at[d - 1],
                    send_sem=send_sems.at[d - 1],
                    recv_sem=recv_sems.at[d - 1],
                    device_id=((my + d) % N_DEV,),
                    device_id_type=pl.DeviceIdType.MESH,
                )
                rdma.start()
                descs.append(rdma)
            else:
                amax = jnp.maximum(amax, jnp.max(jnp.abs(blk)))
                ov[pl.ds(my * m_per, m_per), :] = blk

        col = lax.broadcasted_iota(jnp.int32, amax_send.shape, 1)
        msg = jnp.full(amax_send.shape, amax, jnp.float32)
        for d in range(1, N_DEV):
            msg = jnp.where(col == d, blkmax[d], msg)
        amax_send[...] = msg
        am_descs = []
        for d in range(1, N_DEV):
            rdma = pltpu.make_async_remote_copy(
                src_ref=amax_send,
                dst_ref=amax_recv.at[d - 1],
                send_sem=am_send_sems.at[d - 1],
                recv_sem=am_recv_sems.at[d - 1],
                device_id=((my + d) % N_DEV,),
                device_id_type=pl.DeviceIdType.MESH,
            )
            rdma.start()
            am_descs.append(rdma)
        for am in am_descs:
            am.wait_recv()

        gmax = jnp.maximum(amax, jnp.max(amax_recv[...]))
        scale = gmax / 448.0
        inv = 448.0 / gmax

        own = ov[pl.ds(my * m_per, m_per), :]
        ov[pl.ds(my * m_per, m_per), :] = (
            (own * inv).astype(jnp.float8_e4m3fn).astype(jnp.float32) * scale
        )
        flush(my)

        for t in range(N_DEV - 1):
            d = offs[t]
            descs[t].wait_recv()
            src = (my - d) % N_DEV
            bm = jnp.max(amax_recv[d - 1, :, d])
            val = iin_ref[d - 1].astype(jnp.float32) * (bm / I16MAX)
            ov[pl.ds(src * m_per, m_per), :] = (
                (val * inv).astype(jnp.float8_e4m3fn).astype(jnp.float32)
                * scale
            )
            flush(src)

        for cp in out_dmas:
            cp.wait()
        for rdma in descs + am_descs:
            rdma.wait_send()

    out_space = (
        pltpu.MemorySpace.HBM if OUT_HBM else pltpu.MemorySpace.VMEM
    )
    grid_spec = pltpu.PrefetchScalarGridSpec(
        num_scalar_prefetch=0,
        in_specs=[
            pl.BlockSpec(memory_space=pltpu.MemorySpace.VMEM),
            pl.BlockSpec(memory_space=pltpu.MemorySpace.HBM),
        ],
        out_specs=pl.BlockSpec(memory_space=out_space),
        scratch_shapes=[
            pltpu.VMEM(
                (N_DEV * m_per if OUT_HBM else 8, n_per if OUT_HBM else 128),
                jnp.float32,
            ),
            pltpu.VMEM((2, k, n_per), jnp.float32),
            pltpu.VMEM((N_DEV - 1, m_per, n_per), jnp.int16),
            pltpu.VMEM((N_DEV - 1, m_per, n_per), jnp.int16),
            pltpu.VMEM((8, 128), jnp.float32),
            pltpu.VMEM((N_DEV - 1, 8, 128), jnp.float32),
            pltpu.SemaphoreType.DMA((2,)),
            pltpu.SemaphoreType.DMA((N_DEV,)),
            pltpu.SemaphoreType.DMA((N_DEV - 1,)),
            pltpu.SemaphoreType.DMA((N_DEV - 1,)),
            pltpu.SemaphoreType.DMA((N_DEV - 1,)),
            pltpu.SemaphoreType.DMA((N_DEV - 1,)),
        ],
    )
    return pl.pallas_call(
        body,
        out_shape=jax.ShapeDtypeStruct((N_DEV * m_per, n_per), jnp.float32),
        grid_spec=grid_spec,
        compiler_params=pltpu.CompilerParams(
            collective_id=0, vmem_limit_bytes=100 * 1024 * 1024
        ),
    )(x, w_mat)
